# Initial kernel scaffold; baseline (speedup 1.0000x reference)
import jax
import jax.numpy as jnp
from jax.experimental import pallas as pl


def kernel(session_idx, session_weight, channel_embed):
    raise NotImplementedError("write your pallas kernel here")


# speedup vs baseline: 1.8870x; 1.8870x over previous
import functools
import jax
import jax.numpy as jnp
from jax import lax
from jax.experimental import pallas as pl
from jax.experimental.pallas import tpu as pltpu
from jax.experimental.pallas import tpu_sc as plsc

N_SESSIONS = 1000
D_MODEL = 128
N_CHANNELS = 128
BATCH = 4096

NC, NS = 2, 16
NW = NC * NS
BPW = BATCH // NW
CH = 2
NCHUNK = BPW // CH

_mesh = plsc.VectorSubcoreMesh(
    core_axis_name="c", subcore_axis_name="s", num_cores=NC, num_subcores=NS
)


@functools.partial(
    pl.kernel,
    out_type=(
        jax.ShapeDtypeStruct((BATCH, D_MODEL), jnp.float32),
        jax.ShapeDtypeStruct((BATCH, N_CHANNELS, D_MODEL), jnp.float32),
    ),
    mesh=_mesh,
    scratch_types=[
        pltpu.VMEM((BPW,), jnp.int32),
        pltpu.VMEM((NCHUNK, CH), jnp.int32),
        pltpu.VMEM((BPW, D_MODEL), jnp.float32),
        pltpu.VMEM((CH, N_CHANNELS, D_MODEL), jnp.float32),
        pltpu.VMEM((CH, N_CHANNELS, D_MODEL), jnp.float32),
        pltpu.SemaphoreType.DMA,
        pltpu.SemaphoreType.DMA,
        pltpu.SemaphoreType.DMA,
        pltpu.SemaphoreType.DMA,
        pltpu.SemaphoreType.DMA,
    ],
)
def _dual_gather(idx_flat_hbm, idx_ch_hbm, sess_hbm, chan_hbm,
                 out1_hbm, out2_hbm,
                 idx_flat_v, idx_ch_v, sess_rows_v, buf0, buf1,
                 sess_sem, g0, g1, s0, s1):
    wid = lax.axis_index("s") * NC + lax.axis_index("c")
    base = wid * BPW

    pltpu.sync_copy(idx_flat_hbm.at[wid], idx_flat_v)
    pltpu.sync_copy(idx_ch_hbm.at[wid], idx_ch_v)

    sess_cp = pltpu.async_copy(sess_hbm.at[idx_flat_v], sess_rows_v, sess_sem)

    bufs = (buf0, buf1)
    gsems = (g0, g1)
    ssems = (s0, s1)

    def gather(j, b):
        pltpu.async_copy(chan_hbm.at[idx_ch_v.at[j]], bufs[b], gsems[b])

    def wait_gather(j, b):
        pltpu.make_async_copy(chan_hbm.at[idx_ch_v.at[j]], bufs[b], gsems[b]).wait()

    def scatter(j, b):
        pltpu.async_copy(bufs[b], out2_hbm.at[pl.ds(base + j * CH, CH)], ssems[b])

    def wait_scatter(b):
        pltpu.make_async_copy(
            bufs[b], out2_hbm.at[pl.ds(base, CH)], ssems[b]
        ).wait()

    gather(0, 0)
    gather(1, 1)
    wait_gather(0, 0)
    scatter(0, 0)
    wait_gather(1, 1)
    scatter(1, 1)

    def body(g, carry):
        j0 = 2 * g
        wait_scatter(0)
        gather(j0, 0)
        wait_scatter(1)
        gather(j0 + 1, 1)
        wait_gather(j0, 0)
        scatter(j0, 0)
        wait_gather(j0 + 1, 1)
        scatter(j0 + 1, 1)
        return carry

    lax.fori_loop(1, NCHUNK // 2, body, 0)

    wait_scatter(0)
    wait_scatter(1)
    sess_cp.wait()
    pltpu.sync_copy(sess_rows_v, out1_hbm.at[pl.ds(base, BPW)])


def kernel(session_idx, session_weight, channel_embed):
    idx = session_idx.astype(jnp.int32)
    idx_flat = idx.reshape(NW, BPW)
    idx_ch = idx.reshape(NW, NCHUNK, CH)
    out1, out2 = _dual_gather(idx_flat, idx_ch, session_weight, channel_embed)
    return (out1, out2)


# speedup vs baseline: 1.9228x; 1.0190x over previous
import functools
import jax
import jax.numpy as jnp
from jax import lax
from jax.experimental import pallas as pl
from jax.experimental.pallas import tpu as pltpu
from jax.experimental.pallas import tpu_sc as plsc

N_SESSIONS = 1000
D_MODEL = 128
N_CHANNELS = 128
BATCH = 4096

NC, NS = 2, 16
NW = NC * NS
BPW = BATCH // NW
CH = 1
NB = 4
NCHUNK = BPW // CH

_mesh = plsc.VectorSubcoreMesh(
    core_axis_name="c", subcore_axis_name="s", num_cores=NC, num_subcores=NS
)


@functools.partial(
    pl.kernel,
    out_type=(
        jax.ShapeDtypeStruct((BATCH, D_MODEL), jnp.float32),
        jax.ShapeDtypeStruct((BATCH, N_CHANNELS, D_MODEL), jnp.float32),
    ),
    mesh=_mesh,
    scratch_types=[
        pltpu.VMEM((BPW,), jnp.int32),
        pltpu.VMEM((NCHUNK, CH), jnp.int32),
        pltpu.VMEM((BPW, D_MODEL), jnp.float32),
        [pltpu.VMEM((CH, N_CHANNELS, D_MODEL), jnp.float32) for _ in range(NB)],
        pltpu.SemaphoreType.DMA,
        [pltpu.SemaphoreType.DMA for _ in range(NB)],
        [pltpu.SemaphoreType.DMA for _ in range(NB)],
    ],
)
def _dual_gather(idx_flat_hbm, idx_ch_hbm, sess_hbm, chan_hbm,
                 out1_hbm, out2_hbm,
                 idx_flat_v, idx_ch_v, sess_rows_v, bufs,
                 sess_sem, gsems, ssems):
    wid = lax.axis_index("s") * NC + lax.axis_index("c")
    base = wid * BPW

    pltpu.sync_copy(idx_flat_hbm.at[wid], idx_flat_v)
    pltpu.sync_copy(idx_ch_hbm.at[wid], idx_ch_v)

    sess_cp = pltpu.async_copy(sess_hbm.at[idx_flat_v], sess_rows_v, sess_sem)

    def gather(j, b):
        pltpu.async_copy(chan_hbm.at[idx_ch_v.at[j]], bufs[b], gsems[b])

    def wait_gather(j, b):
        pltpu.make_async_copy(chan_hbm.at[idx_ch_v.at[j]], bufs[b], gsems[b]).wait()

    def scatter(j, b):
        pltpu.async_copy(bufs[b], out2_hbm.at[pl.ds(base + j * CH, CH)], ssems[b])

    def wait_scatter(b):
        pltpu.make_async_copy(
            bufs[b], out2_hbm.at[pl.ds(base, CH)], ssems[b]
        ).wait()

    for b in range(NB):
        gather(b, b)
    for b in range(NB):
        wait_gather(b, b)
        scatter(b, b)

    def body(g, carry):
        j0 = NB * g
        for b in range(NB):
            wait_scatter(b)
            gather(j0 + b, b)
        for b in range(NB):
            wait_gather(j0 + b, b)
            scatter(j0 + b, b)
        return carry

    lax.fori_loop(1, NCHUNK // NB, body, 0)

    for b in range(NB):
        wait_scatter(b)
    sess_cp.wait()
    pltpu.sync_copy(sess_rows_v, out1_hbm.at[pl.ds(base, BPW)])


def kernel(session_idx, session_weight, channel_embed):
    idx = session_idx.astype(jnp.int32)
    idx_flat = idx.reshape(NW, BPW)
    idx_ch = idx.reshape(NW, NCHUNK, CH)
    out1, out2 = _dual_gather(idx_flat, idx_ch, session_weight, channel_embed)
    return (out1, out2)
